# baseline (speedup 1.0000x reference)
import functools

import jax
import jax.numpy as jnp
from jax import lax
from jax.experimental import pallas as pl
from jax.experimental.pallas import tpu as pltpu
from jax.experimental.pallas import tpu_sc as plsc

_D_MODEL = 64
_SCALE = 8.0
_NUM_WORKERS = 32
_CHUNK = 512


@functools.partial(jax.jit, static_argnames=())
def _embed(xf, table):
    B = xf.shape[0]
    D = table.shape[1]
    b_per_w = B // _NUM_WORKERS
    n_chunks = b_per_w // _CHUNK

    mesh = plsc.VectorSubcoreMesh(core_axis_name="c", subcore_axis_name="s")

    @functools.partial(
        pl.kernel,
        out_type=jax.ShapeDtypeStruct((B, D), jnp.float32),
        mesh=mesh,
        scratch_types=[
            pltpu.VMEM((_CHUNK,), jnp.int32),
            pltpu.VMEM((_CHUNK, D), jnp.float32),
            pltpu.SemaphoreType.DMA,
        ],
        compiler_params=pltpu.CompilerParams(use_tc_tiling_on_sc=False),
    )
    def k(idx_hbm, table_hbm, out_hbm, idx_v, rows_v, sem):
        wid = lax.axis_index("s") * 2 + lax.axis_index("c")
        base = wid * b_per_w

        @pl.loop(0, n_chunks)
        def _chunk_loop(g):
            off = base + g * _CHUNK
            pltpu.sync_copy(idx_hbm.at[pl.ds(off, _CHUNK)], idx_v)
            pltpu.async_copy(table_hbm.at[idx_v], rows_v, sem).wait()

            @pl.loop(0, _CHUNK)
            def _row_loop(r):
                for d in range(D // 16):
                    sl = pl.ds(d * 16, 16)
                    rows_v[r, sl] = rows_v[r, sl] * _SCALE

            pltpu.sync_copy(rows_v, out_hbm.at[pl.ds(off, _CHUNK)])

    return k(xf, table)


def kernel(x, table):
    xf = x.reshape(-1).astype(jnp.int32)
    out = _embed(xf, table)
    return out.reshape(x.shape + (table.shape[1],))


# speedup vs baseline: 1.1405x; 1.1405x over previous
import functools

import jax
import jax.numpy as jnp
from jax import lax
from jax.experimental import pallas as pl
from jax.experimental.pallas import tpu as pltpu
from jax.experimental.pallas import tpu_sc as plsc

_D_MODEL = 64
_SCALE = 8.0
_NUM_WORKERS = 32
_CHUNK = 400
_NBUF = 4


@jax.jit
def _embed(xf, table):
    B = xf.shape[0]
    D = table.shape[1]
    b_per_w = B // _NUM_WORKERS
    n_chunks = b_per_w // _CHUNK
    assert n_chunks % _NBUF == 0

    mesh = plsc.VectorSubcoreMesh(core_axis_name="c", subcore_axis_name="s")

    scratch = (
        [pltpu.VMEM((_CHUNK,), jnp.int32) for _ in range(_NBUF)]
        + [pltpu.VMEM((_CHUNK, D), jnp.float32) for _ in range(_NBUF)]
        + [pltpu.SemaphoreType.DMA for _ in range(3 * _NBUF)]
    )

    @functools.partial(
        pl.kernel,
        out_type=jax.ShapeDtypeStruct((B, D), jnp.float32),
        mesh=mesh,
        scratch_types=scratch,
        compiler_params=pltpu.CompilerParams(use_tc_tiling_on_sc=False),
    )
    def k(idx_hbm, table_hbm, out_hbm, *refs):
        idxs = refs[0:_NBUF]
        rows = refs[_NBUF : 2 * _NBUF]
        gsem = refs[2 * _NBUF : 3 * _NBUF]
        ssem = refs[3 * _NBUF : 4 * _NBUF]
        isem = refs[4 * _NBUF : 5 * _NBUF]

        wid = lax.axis_index("s") * 2 + lax.axis_index("c")
        base = wid * b_per_w

        pltpu.sync_copy(idx_hbm.at[pl.ds(base, _CHUNK)], idxs[0])
        pltpu.async_copy(table_hbm.at[idxs[0]], rows[0], gsem[0])
        pltpu.async_copy(idx_hbm.at[pl.ds(base + _CHUNK, _CHUNK)], idxs[1], isem[1])

        @pl.loop(0, n_chunks, step=_NBUF)
        def _g_loop(g):
            for p in range(_NBUF):
                t = g + p
                c = p
                nx = (p + 1) % _NBUF
                n2 = (p + 2) % _NBUF

                @pl.when(t >= _NBUF - 1)
                def _():
                    pltpu.make_async_copy(
                        rows[nx], out_hbm.at[pl.ds(base, _CHUNK)], ssem[nx]
                    ).wait()

                @pl.when(t + 1 < n_chunks)
                def _():
                    pltpu.make_async_copy(
                        idx_hbm.at[pl.ds(base, _CHUNK)], idxs[nx], isem[nx]
                    ).wait()
                    pltpu.async_copy(table_hbm.at[idxs[nx]], rows[nx], gsem[nx])

                pltpu.make_async_copy(
                    table_hbm.at[idxs[c]], rows[c], gsem[c]
                ).wait()

                @pl.when(t + 2 < n_chunks)
                def _():
                    pltpu.async_copy(
                        idx_hbm.at[pl.ds(base + (t + 2) * _CHUNK, _CHUNK)],
                        idxs[n2],
                        isem[n2],
                    )

                @pl.loop(0, _CHUNK, unroll=8)
                def _row_loop(r):
                    for d in range(D // 16):
                        sl = pl.ds(d * 16, 16)
                        rows[c][r, sl] = rows[c][r, sl] * _SCALE

                pltpu.async_copy(
                    rows[c], out_hbm.at[pl.ds(base + t * _CHUNK, _CHUNK)], ssem[c]
                )

        for q in range(_NBUF - 1, 0, -1):
            b = (n_chunks - q) % _NBUF
            pltpu.make_async_copy(
                rows[b], out_hbm.at[pl.ds(base, _CHUNK)], ssem[b]
            ).wait()

    return k(xf, table)


def kernel(x, table):
    xf = x.reshape(-1).astype(jnp.int32)
    out = _embed(xf, table)
    return out.reshape(x.shape + (table.shape[1],))


# speedup vs baseline: 1.1411x; 1.0006x over previous
import functools

import jax
import jax.numpy as jnp
from jax import lax
from jax.experimental import pallas as pl
from jax.experimental.pallas import tpu as pltpu
from jax.experimental.pallas import tpu_sc as plsc

_D_MODEL = 64
_SCALE = 8.0
_NUM_WORKERS = 32
_CHUNK = 400
_NBUF = 4


def _embed(xf, table, R, S):
    B = xf.shape[0]
    D = table.shape[1]
    b_per_w = B // _NUM_WORKERS
    n_chunks = b_per_w // _CHUNK
    rows_per_chunk = _CHUNK // S
    assert n_chunks % _NBUF == 0 and _CHUNK % S == 0

    mesh = plsc.VectorSubcoreMesh(core_axis_name="c", subcore_axis_name="s")

    scratch = (
        [pltpu.VMEM((_CHUNK,), jnp.int32) for _ in range(_NBUF)]
        + [pltpu.VMEM((_CHUNK, D), jnp.float32) for _ in range(_NBUF)]
        + [pltpu.SemaphoreType.DMA for _ in range(3 * _NBUF)]
    )

    @functools.partial(
        pl.kernel,
        out_type=jax.ShapeDtypeStruct((R, S, D), jnp.float32),
        mesh=mesh,
        scratch_types=scratch,
        compiler_params=pltpu.CompilerParams(use_tc_tiling_on_sc=False),
    )
    def k(idx_hbm, table_hbm, out_hbm, *refs):
        idxs = refs[0:_NBUF]
        rows = refs[_NBUF : 2 * _NBUF]
        gsem = refs[2 * _NBUF : 3 * _NBUF]
        ssem = refs[3 * _NBUF : 4 * _NBUF]
        isem = refs[4 * _NBUF : 5 * _NBUF]

        wid = lax.axis_index("s") * 2 + lax.axis_index("c")
        base = wid * b_per_w
        base_row = wid * (R // _NUM_WORKERS)

        def fire_store(t, c):
            for j in range(rows_per_chunk):
                pltpu.async_copy(
                    rows[c].at[pl.ds(j * S, S)],
                    out_hbm.at[base_row + t * rows_per_chunk + j],
                    ssem[c],
                )

        def drain_store(c):
            for j in range(rows_per_chunk):
                pltpu.make_async_copy(
                    rows[c].at[pl.ds(j * S, S)], out_hbm.at[base_row], ssem[c]
                ).wait()

        pltpu.sync_copy(idx_hbm.at[pl.ds(base, _CHUNK)], idxs[0])
        pltpu.async_copy(table_hbm.at[idxs[0]], rows[0], gsem[0])
        pltpu.async_copy(idx_hbm.at[pl.ds(base + _CHUNK, _CHUNK)], idxs[1], isem[1])

        @pl.loop(0, n_chunks, step=_NBUF)
        def _g_loop(g):
            for p in range(_NBUF):
                t = g + p
                c = p
                nx = (p + 1) % _NBUF
                n2 = (p + 2) % _NBUF

                @pl.when(t >= _NBUF - 1)
                def _():
                    drain_store(nx)

                @pl.when(t + 1 < n_chunks)
                def _():
                    pltpu.make_async_copy(
                        idx_hbm.at[pl.ds(base, _CHUNK)], idxs[nx], isem[nx]
                    ).wait()
                    pltpu.async_copy(table_hbm.at[idxs[nx]], rows[nx], gsem[nx])

                pltpu.make_async_copy(
                    table_hbm.at[idxs[c]], rows[c], gsem[c]
                ).wait()

                @pl.when(t + 2 < n_chunks)
                def _():
                    pltpu.async_copy(
                        idx_hbm.at[pl.ds(base + (t + 2) * _CHUNK, _CHUNK)],
                        idxs[n2],
                        isem[n2],
                    )

                @pl.loop(0, _CHUNK, unroll=8)
                def _row_loop(r):
                    for d in range(D // 16):
                        sl = pl.ds(d * 16, 16)
                        rows[c][r, sl] = rows[c][r, sl] * _SCALE

                fire_store(t, c)

        for q in range(_NBUF - 1, 0, -1):
            drain_store((n_chunks - q) % _NBUF)

    return k(xf, table)


def kernel(x, table):
    xf = x.reshape(-1).astype(jnp.int32)
    return _embed(xf, table, x.shape[0], x.shape[1])
